# baseline (speedup 1.0000x reference)
import functools

import jax
import jax.numpy as jnp
from jax import lax
from jax.experimental import pallas as pl
from jax.experimental.pallas import tpu as pltpu
from jax.experimental.pallas import tpu_sc as plsc

N = 10000
NPAD = 10240
D = 128
E = 320000
EPAD = 327680
NW = 32
NS = 16
B = 128
NB = EPAD // (NW * B)
RPT = NPAD // NS

_mesh = plsc.VectorSubcoreMesh(core_axis_name="c", subcore_axis_name="s")


HR = NPAD // 128
HRT = HR // NS


@functools.partial(
    pl.kernel,
    out_type=jax.ShapeDtypeStruct((2, HR, 128), jnp.float32),
    mesh=_mesh,
    compiler_params=pltpu.CompilerParams(needs_layout_passes=False),
    scratch_types=[
        pltpu.VMEM((NB, B), jnp.int32),
        pltpu.VMEM((HR, 128), jnp.float32),
        pltpu.VMEM((NS * 8, 128), jnp.float32),
        pltpu.VMEM((8, 128), jnp.float32),
        pltpu.VMEM_SHARED((NS, HR, 128), jnp.float32),
    ],
)
def _sc_deg(cidx_hbm, out_hbm, cidx_v, hist, rbuf, obuf, stage):
    c = lax.axis_index("c")
    s = lax.axis_index("s")
    wid = c * NS + s

    def zb(r, carry):
        for l in range(8):
            hist[r, pl.ds(l * 16, 16)] = jnp.zeros((16,), jnp.float32)
        return carry

    lax.fori_loop(0, HR, zb, 0)
    pltpu.sync_copy(cidx_hbm.at[wid], cidx_v)

    ones = jnp.ones((16,), jnp.float32)

    def body(j, carry):
        r = j // 8
        l = j % 8
        iv = cidx_v[r, pl.ds(l * 16, 16)]
        plsc.addupdate_scatter(hist, (iv >> 7, iv & 127), ones)
        return carry

    lax.fori_loop(0, NB * 8, body, 0)
    pltpu.sync_copy(hist, stage.at[s])
    plsc.subcore_barrier()

    @pl.when(s < HR // 8)
    def _():
        base = s * 8
        for t in range(NS):
            pltpu.sync_copy(stage.at[t, pl.ds(base, 8)],
                            rbuf.at[pl.ds(t * 8, 8)])

        def red(p, carry):
            r = p // 8
            l = p % 8
            acc = rbuf[r, pl.ds(l * 16, 16)]
            for t in range(1, NS):
                acc = acc + rbuf[t * 8 + r, pl.ds(l * 16, 16)]
            obuf[r, pl.ds(l * 16, 16)] = acc
            return carry

        lax.fori_loop(0, 64, red, 0)
        pltpu.sync_copy(obuf, out_hbm.at[c, pl.ds(base, 8)])


K = 2
D2 = D // 2
EPB = 80
NBT = EPAD // (NW * EPB)


@functools.partial(
    pl.kernel,
    out_type=jax.ShapeDtypeStruct((2, NPAD, D), jnp.float32),
    mesh=_mesh,
    compiler_params=pltpu.CompilerParams(use_tc_tiling_on_sc=False),
    scratch_types=[
        pltpu.VMEM((NBT, EPB), jnp.int32),
        pltpu.VMEM((NBT, EPB), jnp.int32),
        pltpu.VMEM((K * EPB, D), jnp.float32),
        pltpu.SemaphoreType.DMA((K,)),
        pltpu.SemaphoreType.DMA((K,)),
        pltpu.VMEM_SHARED((NPAD, D), jnp.float32),
    ],
)
def _sc_agg(y_hbm, ridx_hbm, cidx_hbm, zeros_hbm, out_hbm,
            ridx_v, cidx_v, gring, gsem, ssem, acc):
    gbufs = [gring.at[pl.ds(b * EPB, EPB)] for b in range(K)]
    gsems = [gsem.at[b] for b in range(K)]
    ssems = [ssem.at[b] for b in range(K)]
    c = lax.axis_index("c")
    s = lax.axis_index("s")
    wid = c * NS + s
    base = s * RPT
    pltpu.sync_copy(zeros_hbm, acc.at[pl.ds(base, RPT)])
    pltpu.sync_copy(ridx_hbm.at[wid], ridx_v)
    pltpu.sync_copy(cidx_hbm.at[wid], cidx_v)
    plsc.subcore_barrier()

    for b in range(K):
        pltpu.async_copy(y_hbm.at[ridx_v.at[b]], gbufs[b], gsems[b])

    def body(g, carry):
        j0 = g * K
        for b in range(K):
            j = j0 + b
            pltpu.make_async_copy(y_hbm.at[ridx_v.at[j]], gbufs[b],
                                  gsems[b]).wait()
            pltpu.async_copy(gbufs[b], acc.at[cidx_v.at[j]], ssems[b],
                             add=True)
        for b in range(K):
            jn = j0 + K + b
            @pl.when(jn < NBT)
            def _():
                pltpu.make_async_copy(gbufs[b], acc.at[cidx_v.at[j0 + b]],
                                      ssems[b]).wait()
                pltpu.async_copy(y_hbm.at[ridx_v.at[jn]], gbufs[b], gsems[b])
        return carry

    lax.fori_loop(0, NBT // K, body, 0)
    for b in range(K):
        pltpu.make_async_copy(gbufs[b], acc.at[cidx_v.at[NBT - K + b]],
                              ssems[b]).wait()
    plsc.subcore_barrier()
    pltpu.sync_copy(acc.at[pl.ds(base, RPT)], out_hbm.at[c, pl.ds(base, RPT)])


def _dinv(deg_ref):
    return lax.rsqrt(deg_ref[0] + deg_ref[1] + 1.0)


def _tc1_body(deg_ref, x_ref, w1t_ref, y_ref):
    xw = jnp.dot(x_ref[...], w1t_ref[...], preferred_element_type=jnp.float32)
    y_ref[...] = xw * _dinv(deg_ref)


def _tc2_body(deg_ref, acc_ref, y1_ref, b1_ref, w2t_ref, y2_ref):
    dinv = _dinv(deg_ref)
    pre = (acc_ref[0] + acc_ref[1] + y1_ref[...]) * dinv + b1_ref[...]
    h = jnp.maximum(pre, 0.0)
    y2_ref[...] = jnp.dot(h, w2t_ref[...], preferred_element_type=jnp.float32) * dinv


def _tc3_body(deg_ref, acc_ref, y2_ref, b2_ref, out_ref):
    out_ref[...] = ((acc_ref[0] + acc_ref[1] + y2_ref[...]) * _dinv(deg_ref)
                    + b2_ref[...])


GB = 8
BR = NPAD // GB

_deg_bs = pl.BlockSpec((2, BR, 1), lambda i: (0, i, 0))
_row_bs = pl.BlockSpec((BR, D), lambda i: (i, 0))
_acc_bs = pl.BlockSpec((2, BR, D), lambda i: (0, i, 0))
_w_bs = pl.BlockSpec((D, D), lambda i: (0, 0))
_b_bs = pl.BlockSpec((1, D), lambda i: (0, 0))

_tc1 = pl.pallas_call(
    _tc1_body, grid=(GB,),
    in_specs=[_deg_bs, _row_bs, _w_bs],
    out_specs=_row_bs,
    out_shape=jax.ShapeDtypeStruct((NPAD, D), jnp.float32))
_tc2 = pl.pallas_call(
    _tc2_body, grid=(GB,),
    in_specs=[_deg_bs, _acc_bs, _row_bs, _b_bs, _w_bs],
    out_specs=_row_bs,
    out_shape=jax.ShapeDtypeStruct((NPAD, D), jnp.float32))
_tc3 = pl.pallas_call(
    _tc3_body, grid=(GB,),
    in_specs=[_deg_bs, _acc_bs, _row_bs, _b_bs],
    out_specs=_row_bs,
                      out_shape=jax.ShapeDtypeStruct((NPAD, D), jnp.float32))


def kernel(x, edge_index, W1, b1, W2, b2):
    row = edge_index[0].astype(jnp.int32)
    col = edge_index[1].astype(jnp.int32)
    pad = EPAD - E
    row_p = jnp.concatenate([row, jnp.zeros((pad,), jnp.int32)])
    col_p = jnp.concatenate([col, jnp.full((pad,), N, jnp.int32)])
    ridx = row_p.reshape(NW, NBT, EPB)
    cidx = col_p.reshape(NW, NBT, EPB)
    cidx_deg = col_p.reshape(NW, NB, B)
    xp = jnp.pad(x.astype(jnp.float32), ((0, NPAD - N), (0, 0)))
    w1t = W1.astype(jnp.float32).T
    w2t = W2.astype(jnp.float32).T
    b1r = b1.astype(jnp.float32).reshape(1, D)
    b2r = b2.astype(jnp.float32).reshape(1, D)
    zerosD = jnp.zeros((RPT, D), jnp.float32)

    deg = _sc_deg(cidx_deg).reshape(2, NPAD, 1)
    y1 = _tc1(deg, xp, w1t)
    acc1 = _sc_agg(y1, ridx, cidx, zerosD)
    y2 = _tc2(deg, acc1, y1, b1r, w2t)
    acc2 = _sc_agg(y2, ridx, cidx, zerosD)
    out = _tc3(deg, acc2, y2, b2r)
    return out[:N]


# speedup vs baseline: 1.1304x; 1.1304x over previous
import functools

import jax
import jax.numpy as jnp
from jax import lax
from jax.experimental import pallas as pl
from jax.experimental.pallas import tpu as pltpu
from jax.experimental.pallas import tpu_sc as plsc

N = 10000
NPAD = 10240
D = 128
E = 320000
EPAD = 327680
NW = 32
NS = 16
B = 128
NB = EPAD // (NW * B)
RPT = NPAD // NS

_mesh = plsc.VectorSubcoreMesh(core_axis_name="c", subcore_axis_name="s")

HR = NPAD // 128
HRT = HR // NS


@functools.partial(
    pl.kernel,
    out_type=jax.ShapeDtypeStruct((2, HR, 128), jnp.float32),
    mesh=_mesh,
    compiler_params=pltpu.CompilerParams(needs_layout_passes=False),
    scratch_types=[
        pltpu.VMEM((NB, B), jnp.int32),
        pltpu.VMEM((HR, 128), jnp.float32),
        pltpu.VMEM((NS * 8, 128), jnp.float32),
        pltpu.VMEM((8, 128), jnp.float32),
        pltpu.VMEM_SHARED((NS, HR, 128), jnp.float32),
    ],
)
def _sc_deg(cidx_hbm, out_hbm, cidx_v, hist, rbuf, obuf, stage):
    c = lax.axis_index("c")
    s = lax.axis_index("s")
    wid = c * NS + s

    def zb(r, carry):
        for l in range(8):
            hist[r, pl.ds(l * 16, 16)] = jnp.zeros((16,), jnp.float32)
        return carry

    lax.fori_loop(0, HR, zb, 0)
    pltpu.sync_copy(cidx_hbm.at[wid], cidx_v)

    ones = jnp.ones((16,), jnp.float32)

    def body(j, carry):
        r = j // 8
        l = j % 8
        iv = cidx_v[r, pl.ds(l * 16, 16)]
        plsc.addupdate_scatter(hist, (iv >> 7, iv & 127), ones)
        return carry

    lax.fori_loop(0, NB * 8, body, 0)
    pltpu.sync_copy(hist, stage.at[s])
    plsc.subcore_barrier()

    @pl.when(s < HR // 8)
    def _():
        base = s * 8
        for t in range(NS):
            pltpu.sync_copy(stage.at[t, pl.ds(base, 8)],
                            rbuf.at[pl.ds(t * 8, 8)])

        def red(p, carry):
            r = p // 8
            l = p % 8
            acc = rbuf[r, pl.ds(l * 16, 16)]
            for t in range(1, NS):
                acc = acc + rbuf[t * 8 + r, pl.ds(l * 16, 16)]
            obuf[r, pl.ds(l * 16, 16)] = acc
            return carry

        lax.fori_loop(0, 64, red, 0)
        pltpu.sync_copy(obuf, out_hbm.at[c, pl.ds(base, 8)])


K = 3
D2 = D // 2
DW = D2 // 2
NB2 = 162
EPAD2 = NS * NB2 * B


@functools.partial(
    pl.kernel,
    out_type=jax.ShapeDtypeStruct((2, NPAD, D2), jnp.float32),
    mesh=_mesh,
    compiler_params=pltpu.CompilerParams(use_tc_tiling_on_sc=False,
                                         needs_layout_passes=False),
    scratch_types=[
        pltpu.VMEM((NB2, B), jnp.int32),
        pltpu.VMEM((NB2, B), jnp.int32),
        pltpu.VMEM((B, DW), jnp.int32),
        pltpu.VMEM((B, DW), jnp.int32),
        pltpu.VMEM((B, DW), jnp.int32),
        pltpu.VMEM((B, D2), jnp.float32),
        pltpu.VMEM((B, D2), jnp.float32),
        pltpu.VMEM((B, D2), jnp.float32),
        pltpu.SemaphoreType.DMA((K,)),
        pltpu.SemaphoreType.DMA((K,)),
        pltpu.VMEM_SHARED((NPAD, D2), jnp.float32),
    ],
)
def _sc_agg(ybf_hbm, ridx_hbm, cidx_hbm, zeros_hbm, out_hbm,
            ridx_v, cidx_v, ib0, ib1, ib2, fb0, fb1, fb2, gsem, ssem, acc):
    ibufs = [ib0, ib1, ib2]
    fbufs = [fb0, fb1, fb2]
    gsems = [gsem.at[b] for b in range(K)]
    ssems = [ssem.at[b] for b in range(K)]
    c = lax.axis_index("c")
    s = lax.axis_index("s")
    base = s * RPT
    pltpu.sync_copy(zeros_hbm, acc.at[pl.ds(base, RPT)])
    pltpu.sync_copy(ridx_hbm.at[c, s], ridx_v)
    pltpu.sync_copy(cidx_hbm.at[s], cidx_v)
    plsc.subcore_barrier()
    himask = jnp.full((16,), -65536, jnp.int32)

    for b in range(K):
        pltpu.async_copy(ybf_hbm.at[ridx_v.at[b]], ibufs[b], gsems[b])

    def body(g, carry):
        j0 = g * K
        for b in range(K):
            j = j0 + b
            pltpu.make_async_copy(ybf_hbm.at[ridx_v.at[j]], ibufs[b],
                                  gsems[b]).wait()
            ib, fb = ibufs[b], fbufs[b]

            def widen(r, carry2):
                for g2 in range(2):
                    w = ib[r, pl.ds(16 * g2, 16)]
                    fb[r, pl.ds(32 * g2, 16)] = plsc.bitcast(
                        w << 16, jnp.float32)
                    fb[r, pl.ds(32 * g2 + 16, 16)] = plsc.bitcast(
                        w & himask, jnp.float32)
                return carry2

            lax.fori_loop(0, B, widen, 0)
            pltpu.async_copy(fbufs[b], acc.at[cidx_v.at[j]], ssems[b],
                             add=True)
        for b in range(K):
            jn = j0 + K + b

            @pl.when(jn < NB2)
            def _():
                pltpu.make_async_copy(fbufs[b], acc.at[cidx_v.at[j0 + b]],
                                      ssems[b]).wait()
                pltpu.async_copy(ybf_hbm.at[ridx_v.at[jn]], ibufs[b],
                                 gsems[b])
        return carry

    lax.fori_loop(0, NB2 // K, body, 0)
    for b in range(K):
        pltpu.make_async_copy(fbufs[b], acc.at[cidx_v.at[NB2 - K + b]],
                              ssems[b]).wait()
    plsc.subcore_barrier()
    pltpu.sync_copy(acc.at[pl.ds(base, RPT)], out_hbm.at[c, pl.ds(base, RPT)])


def _dinv(deg_ref):
    return lax.rsqrt(deg_ref[0] + deg_ref[1] + 1.0)


def _bf16_bits(a):
    u = jax.lax.bitcast_convert_type(a, jnp.uint32)
    return (u + 0x7FFF + ((u >> 16) & 1)) >> 16


def _pack_half(y, c2):
    words = []
    for g2 in range(2):
        lo = _bf16_bits(y[:, 64 * c2 + 32 * g2: 64 * c2 + 32 * g2 + 16])
        hi = _bf16_bits(y[:, 64 * c2 + 32 * g2 + 16: 64 * c2 + 32 * g2 + 32])
        words.append(jax.lax.bitcast_convert_type(lo | (hi << 16), jnp.int32))
    return jnp.concatenate(words, axis=1)


def _tc1_body(deg_ref, x_ref, w1t_ref, y_ref, ybf_ref):
    xw = jnp.dot(x_ref[...], w1t_ref[...], preferred_element_type=jnp.float32)
    y = xw * _dinv(deg_ref)
    y_ref[...] = y
    ybf_ref[:, 0, :] = _pack_half(y, 0)
    ybf_ref[:, 1, :] = _pack_half(y, 1)


def _tc2_body(deg_ref, acc_ref, y1_ref, b1_ref, w2t_ref, y2_ref, ybf_ref):
    dinv = _dinv(deg_ref)
    accf = jnp.concatenate([acc_ref[0], acc_ref[1]], axis=1)
    pre = (accf + y1_ref[...]) * dinv + b1_ref[...]
    h = jnp.maximum(pre, 0.0)
    y2 = jnp.dot(h, w2t_ref[...], preferred_element_type=jnp.float32) * dinv
    y2_ref[...] = y2
    ybf_ref[:, 0, :] = _pack_half(y2, 0)
    ybf_ref[:, 1, :] = _pack_half(y2, 1)


def _tc3_body(deg_ref, acc_ref, y2_ref, b2_ref, out_ref):
    accf = jnp.concatenate([acc_ref[0], acc_ref[1]], axis=1)
    out_ref[...] = (accf + y2_ref[...]) * _dinv(deg_ref) + b2_ref[...]


GB = 8
BR = NPAD // GB

_deg_bs = pl.BlockSpec((2, BR, 1), lambda i: (0, i, 0))
_row_bs = pl.BlockSpec((BR, D), lambda i: (i, 0))
_acc_bs = pl.BlockSpec((2, BR, D2), lambda i: (0, i, 0))
_w_bs = pl.BlockSpec((D, D), lambda i: (0, 0))
_b_bs = pl.BlockSpec((1, D), lambda i: (0, 0))

_ybf_bs = pl.BlockSpec((BR, 2, DW), lambda i: (i, 0, 0))
_y_and_bf = [jax.ShapeDtypeStruct((NPAD, D), jnp.float32),
             jax.ShapeDtypeStruct((NPAD, 2, DW), jnp.int32)]

_tc1 = pl.pallas_call(
    _tc1_body, grid=(GB,),
    in_specs=[_deg_bs, _row_bs, _w_bs],
    out_specs=[_row_bs, _ybf_bs],
    out_shape=_y_and_bf)
_tc2 = pl.pallas_call(
    _tc2_body, grid=(GB,),
    in_specs=[_deg_bs, _acc_bs, _row_bs, _b_bs, _w_bs],
    out_specs=[_row_bs, _ybf_bs],
    out_shape=_y_and_bf)
_tc3 = pl.pallas_call(
    _tc3_body, grid=(GB,),
    in_specs=[_deg_bs, _acc_bs, _row_bs, _b_bs],
    out_specs=_row_bs,
    out_shape=jax.ShapeDtypeStruct((NPAD, D), jnp.float32))


def kernel(x, edge_index, W1, b1, W2, b2):
    row = edge_index[0].astype(jnp.int32)
    col = edge_index[1].astype(jnp.int32)
    pad = EPAD2 - E
    row_p = jnp.concatenate([row, jnp.zeros((pad,), jnp.int32)])
    col_p = jnp.concatenate([col, jnp.full((pad,), N, jnp.int32)])
    r3 = row_p.reshape(NS, NB2, B)
    ridx2 = jnp.stack([2 * r3, 2 * r3 + 1])
    cidx2 = col_p.reshape(NS, NB2, B)
    cidx_deg = col_p[:EPAD].reshape(NW, NB, B)
    xp = jnp.pad(x.astype(jnp.float32), ((0, NPAD - N), (0, 0)))
    w1t = W1.astype(jnp.float32).T
    w2t = W2.astype(jnp.float32).T
    b1r = b1.astype(jnp.float32).reshape(1, D)
    b2r = b2.astype(jnp.float32).reshape(1, D)
    zerosH = jnp.zeros((RPT, D2), jnp.float32)

    deg = _sc_deg(cidx_deg).reshape(2, NPAD, 1)
    y1, y1bf = _tc1(deg, xp, w1t)
    acc1 = _sc_agg(y1bf.reshape(2 * NPAD, DW), ridx2, cidx2, zerosH)
    y2, y2bf = _tc2(deg, acc1, y1, b1r, w2t)
    acc2 = _sc_agg(y2bf.reshape(2 * NPAD, DW), ridx2, cidx2, zerosH)
    out = _tc3(deg, acc2, y2, b2r)
    return out[:N]
